# baseline (speedup 1.0000x reference)
import functools

import jax
import jax.numpy as jnp
import numpy as np
from jax import lax
from jax.experimental import pallas as pl
from jax.experimental.pallas import tpu as pltpu
from jax.experimental.pallas import tpu_sc as plsc

B, H, F, D, V = 4096, 50, 26, 32, 100000
NC, NS = 2, 16
NW = NC * NS
P = B * H
PPW = P // NW
CP = 32
NCH = PPW // CP
E = CP * F
NB = 4
GSZ = [128] * 6 + [64]
GOFF = [0, 128, 256, 384, 512, 640, 768]
OR = CP // 4
ORPW = PPW // 4

_OFF_NP = (np.arange(E, dtype=np.int32) % F) * V

_mesh = plsc.VectorSubcoreMesh(
    core_axis_name="c", subcore_axis_name="s", num_cores=NC, num_subcores=NS
)


@functools.partial(
    pl.kernel,
    out_type=jax.ShapeDtypeStruct((P // 4, 128), jnp.float32),
    mesh=_mesh,
    scratch_types=[
        pltpu.VMEM((NB, E), jnp.int32),
        pltpu.VMEM((NB, E), jnp.int32),
        pltpu.VMEM((E,), jnp.int32),
        pltpu.VMEM((E, D), jnp.float32),
        pltpu.VMEM((E, D), jnp.float32),
        pltpu.VMEM((E, D), jnp.float32),
        pltpu.VMEM((E, D), jnp.float32),
        pltpu.VMEM((OR, 128), jnp.float32),
        pltpu.SemaphoreType.DMA,
        pltpu.SemaphoreType.DMA,
        pltpu.SemaphoreType.DMA,
        pltpu.SemaphoreType.DMA,
        pltpu.SemaphoreType.DMA,
        pltpu.SemaphoreType.DMA,
        pltpu.SemaphoreType.DMA,
        pltpu.SemaphoreType.DMA,
    ],
    compiler_params=pltpu.CompilerParams(use_tc_tiling_on_sc=False),
)
def _embed_sum(nf_hbm, w_hbm, off_hbm, out_hbm, idxc_v, fidx_v, off_v,
               rows0_v, rows1_v, rows2_v, rows3_v, outb_v,
               sem0, sem1, sem2, sem3, isem0, isem1, isem2, isem3):
    cid = lax.axis_index("c")
    sid = lax.axis_index("s")
    wid = sid * NC + cid
    wtab = w_hbm.at[0]
    rows = (rows0_v, rows1_v, rows2_v, rows3_v)
    sems = (sem0, sem1, sem2, sem3)
    isems = (isem0, isem1, isem2, isem3)

    pltpu.sync_copy(off_hbm, off_v)

    def idx_fetch(c, buf):
        pltpu.async_copy(
            nf_hbm.at[pl.ds((wid * PPW + c * CP) * F, E)],
            idxc_v.at[buf],
            isems[buf],
        )

    def prep_fire(c, buf):
        pltpu.make_async_copy(
            nf_hbm.at[pl.ds(0, E)], idxc_v.at[buf], isems[buf]
        ).wait()

        def add16(g, carry):
            fidx_v[buf, pl.ds(g * 16, 16)] = (
                idxc_v[buf, pl.ds(g * 16, 16)] + off_v[pl.ds(g * 16, 16)]
            )
            return carry

        lax.fori_loop(0, E // 16, add16, 0)

        @pl.when(c + NB < NCH)
        def _next_idx():
            idx_fetch(c + NB, buf)

        for j, g in enumerate(GSZ):
            pltpu.async_copy(
                wtab.at[fidx_v.at[buf, pl.ds(GOFF[j], g)]],
                rows[buf].at[pl.ds(GOFF[j], g)],
                sems[buf],
            )

    def drain_acc(c, buf):
        pltpu.make_async_copy(wtab.at[pl.ds(0, E)], rows[buf], sems[buf]).wait()
        rv = rows[buf]

        def acc(r, carry):
            for q in range(4):
                base = r * (4 * F) + q * F
                a0 = rv[base, pl.ds(0, 16)]
                a1 = rv[base, pl.ds(16, 16)]
                for i in range(1, F):
                    a0 = a0 + rv[base + i, pl.ds(0, 16)]
                    a1 = a1 + rv[base + i, pl.ds(16, 16)]
                outb_v[r, pl.ds(q * 32, 16)] = a0
                outb_v[r, pl.ds(q * 32 + 16, 16)] = a1
            return carry

        lax.fori_loop(0, OR, acc, 0)
        pltpu.sync_copy(outb_v, out_hbm.at[pl.ds(wid * ORPW + c * OR, OR)])

    for b in range(NB):
        idx_fetch(b, b)
    for b in range(NB - 1):
        prep_fire(b, b)

    def body(k, carry):
        for b in range(NB):
            c = k * NB + b

            @pl.when(c + NB - 1 < NCH)
            def _prefetch():
                prep_fire(c + NB - 1, (b + NB - 1) % NB)

            drain_acc(c, b)
        return carry

    lax.fori_loop(0, NCH // NB, body, 0)


def kernel(node_feature, W):
    nf_flat = node_feature.reshape(-1)
    off = jnp.asarray(_OFF_NP)
    out = _embed_sum(nf_flat, W, off)
    return out.reshape(B, H, D)


# speedup vs baseline: 1.0253x; 1.0253x over previous
import functools

import jax
import jax.numpy as jnp
import numpy as np
from jax import lax
from jax.experimental import pallas as pl
from jax.experimental.pallas import tpu as pltpu
from jax.experimental.pallas import tpu_sc as plsc

B, H, F, D, V = 4096, 50, 26, 32, 100000
NC, NS = 2, 16
NW = NC * NS
P = B * H
PPW = P // NW
CP = 64
NCH = PPW // CP
E = CP * F
NB = 2
GSZ = [128] * 13
GOFF = [j * 128 for j in range(13)]
OR = CP // 4
ORPW = PPW // 4

_OFF_NP = (np.arange(E, dtype=np.int32) % F) * V

_mesh = plsc.VectorSubcoreMesh(
    core_axis_name="c", subcore_axis_name="s", num_cores=NC, num_subcores=NS
)


@functools.partial(
    pl.kernel,
    out_type=jax.ShapeDtypeStruct((P // 4, 128), jnp.float32),
    mesh=_mesh,
    scratch_types=[
        pltpu.VMEM((NB, E), jnp.int32),
        pltpu.VMEM((NB, E), jnp.int32),
        pltpu.VMEM((E,), jnp.int32),
        pltpu.VMEM((E, D), jnp.float32),
        pltpu.VMEM((E, D), jnp.float32),
        pltpu.VMEM((OR, 128), jnp.float32),
        pltpu.SemaphoreType.DMA,
        pltpu.SemaphoreType.DMA,
        pltpu.SemaphoreType.DMA,
        pltpu.SemaphoreType.DMA,
    ],
    compiler_params=pltpu.CompilerParams(use_tc_tiling_on_sc=False),
)
def _embed_sum(nf_hbm, w_hbm, off_hbm, out_hbm, idxc_v, fidx_v, off_v,
               rows0_v, rows1_v, outb_v, sem0, sem1, isem0, isem1):
    cid = lax.axis_index("c")
    sid = lax.axis_index("s")
    wid = sid * NC + cid
    wtab = w_hbm.at[0]
    rows = (rows0_v, rows1_v)
    sems = (sem0, sem1)
    isems = (isem0, isem1)

    pltpu.sync_copy(off_hbm, off_v)

    def idx_fetch(c, buf):
        pltpu.async_copy(
            nf_hbm.at[pl.ds((wid * PPW + c * CP) * F, E)],
            idxc_v.at[buf],
            isems[buf],
        )

    def prep_fire(c, buf):
        pltpu.make_async_copy(
            nf_hbm.at[pl.ds(0, E)], idxc_v.at[buf], isems[buf]
        ).wait()

        def add16(g, carry):
            fidx_v[buf, pl.ds(g * 16, 16)] = (
                idxc_v[buf, pl.ds(g * 16, 16)] + off_v[pl.ds(g * 16, 16)]
            )
            return carry

        lax.fori_loop(0, E // 16, add16, 0)

        @pl.when(c + NB < NCH)
        def _next_idx():
            idx_fetch(c + NB, buf)

        for j, g in enumerate(GSZ):
            pltpu.async_copy(
                wtab.at[fidx_v.at[buf, pl.ds(GOFF[j], g)]],
                rows[buf].at[pl.ds(GOFF[j], g)],
                sems[buf],
            )

    def drain_acc(c, buf):
        pltpu.make_async_copy(wtab.at[pl.ds(0, E)], rows[buf], sems[buf]).wait()
        rv = rows[buf]

        def acc(r, carry):
            for q in range(4):
                base = r * (4 * F) + q * F
                a0 = rv[base, pl.ds(0, 16)]
                a1 = rv[base, pl.ds(16, 16)]
                for i in range(1, F):
                    a0 = a0 + rv[base + i, pl.ds(0, 16)]
                    a1 = a1 + rv[base + i, pl.ds(16, 16)]
                outb_v[r, pl.ds(q * 32, 16)] = a0
                outb_v[r, pl.ds(q * 32 + 16, 16)] = a1
            return carry

        lax.fori_loop(0, OR, acc, 0)
        pltpu.sync_copy(outb_v, out_hbm.at[pl.ds(wid * ORPW + c * OR, OR)])

    for b in range(NB):
        idx_fetch(b, b)
    for b in range(NB - 1):
        prep_fire(b, b)

    def body(k, carry):
        for b in range(NB):
            c = k * NB + b

            @pl.when(c + NB - 1 < NCH)
            def _prefetch():
                prep_fire(c + NB - 1, (b + NB - 1) % NB)

            drain_acc(c, b)
        return carry

    lax.fori_loop(0, NCH // NB, body, 0)


def kernel(node_feature, W):
    nf_flat = node_feature.reshape(-1)
    off = jnp.asarray(_OFF_NP)
    out = _embed_sum(nf_flat, W, off)
    return out.reshape(B, H, D)
